# baseline (speedup 1.0000x reference)
import jax
import jax.numpy as jnp
from jax import lax
from jax.experimental import pallas as pl
from jax.experimental.pallas import tpu as pltpu

VARIANCE_FOCUS = 0.85

_ROWS = 8192
_COLS = 512
_BLK_ROWS = 1024
_GRID = _ROWS // _BLK_ROWS


def _silog_body(est_ref, gt_ref, out_ref, accd_ref, accd2_ref, accn_ref):
    i = pl.program_id(0)

    @pl.when(i == 0)
    def _init():
        accd_ref[...] = jnp.zeros_like(accd_ref)
        accd2_ref[...] = jnp.zeros_like(accd2_ref)
        accn_ref[...] = jnp.zeros_like(accn_ref)

    est = est_ref[...]
    gt = gt_ref[...]
    mask = gt > 1.0
    d = jnp.log(jnp.where(mask, est / gt, 1.0))
    d2 = d * d
    mf = jnp.where(mask, 1.0, 0.0)

    ones = jnp.ones((8, _BLK_ROWS), jnp.float32)
    dims = (((1,), (0,)), ((), ()))
    accd_ref[...] += lax.dot_general(ones, d, dims,
                                     preferred_element_type=jnp.float32)
    accd2_ref[...] += lax.dot_general(ones, d2, dims,
                                      preferred_element_type=jnp.float32)
    accn_ref[...] += lax.dot_general(ones, mf, dims,
                                     preferred_element_type=jnp.float32)

    @pl.when(i == _GRID - 1)
    def _fin():
        sd = jnp.sum(accd_ref[...])
        sd2 = jnp.sum(accd2_ref[...])
        n = jnp.sum(accn_ref[...])
        mean_d = sd / n
        mean_d2 = sd2 / n
        out_ref[0] = jnp.sqrt(mean_d2 - VARIANCE_FOCUS * mean_d * mean_d) * 10.0


def kernel(depth_est, depth_gt):
    est2d = depth_est.reshape(_ROWS, _COLS)
    gt2d = depth_gt.reshape(_ROWS, _COLS)
    out = pl.pallas_call(
        _silog_body,
        grid=(_GRID,),
        in_specs=[
            pl.BlockSpec((_BLK_ROWS, _COLS), lambda i: (i, 0)),
            pl.BlockSpec((_BLK_ROWS, _COLS), lambda i: (i, 0)),
        ],
        out_specs=pl.BlockSpec(memory_space=pltpu.SMEM),
        out_shape=jax.ShapeDtypeStruct((1,), jnp.float32),
        scratch_shapes=[
            pltpu.VMEM((8, _COLS), jnp.float32),
            pltpu.VMEM((8, _COLS), jnp.float32),
            pltpu.VMEM((8, _COLS), jnp.float32),
        ],
    )(est2d, gt2d)
    return out[0]


# speedup vs baseline: 1.0794x; 1.0794x over previous
import jax
import jax.numpy as jnp
from jax import lax
from jax.experimental import pallas as pl
from jax.experimental.pallas import tpu as pltpu

VARIANCE_FOCUS = 0.85

_ROWS = 8192
_COLS = 512
_BLK_ROWS = 2048
_GRID = _ROWS // _BLK_ROWS


def _silog_body(est_ref, gt_ref, out_ref, accd_ref, accd2_ref, accn_ref):
    i = pl.program_id(0)

    @pl.when(i == 0)
    def _init():
        accd_ref[...] = jnp.zeros_like(accd_ref)
        accd2_ref[...] = jnp.zeros_like(accd2_ref)
        accn_ref[...] = jnp.zeros_like(accn_ref)

    est = est_ref[...]
    gt = gt_ref[...]
    mask = gt > 1.0
    d = jnp.log(jnp.where(mask, est, 1.0)) - jnp.log(jnp.where(mask, gt, 1.0))
    d2 = d * d
    mf = jnp.where(mask, 1.0, 0.0)

    ones = jnp.ones((8, _BLK_ROWS), jnp.float32)
    dims = (((1,), (0,)), ((), ()))
    accd_ref[...] += lax.dot_general(ones, d, dims,
                                     preferred_element_type=jnp.float32)
    accd2_ref[...] += lax.dot_general(ones, d2, dims,
                                      preferred_element_type=jnp.float32)
    accn_ref[...] += lax.dot_general(ones, mf, dims,
                                     preferred_element_type=jnp.float32)

    @pl.when(i == _GRID - 1)
    def _fin():
        sd = jnp.sum(accd_ref[...])
        sd2 = jnp.sum(accd2_ref[...])
        n = jnp.sum(accn_ref[...])
        mean_d = sd / n
        mean_d2 = sd2 / n
        out_ref[0] = jnp.sqrt(mean_d2 - VARIANCE_FOCUS * mean_d * mean_d) * 10.0


def kernel(depth_est, depth_gt):
    est2d = depth_est.reshape(_ROWS, _COLS)
    gt2d = depth_gt.reshape(_ROWS, _COLS)
    out = pl.pallas_call(
        _silog_body,
        grid=(_GRID,),
        in_specs=[
            pl.BlockSpec((_BLK_ROWS, _COLS), lambda i: (i, 0)),
            pl.BlockSpec((_BLK_ROWS, _COLS), lambda i: (i, 0)),
        ],
        out_specs=pl.BlockSpec(memory_space=pltpu.SMEM),
        out_shape=jax.ShapeDtypeStruct((1,), jnp.float32),
        scratch_shapes=[
            pltpu.VMEM((8, _COLS), jnp.float32),
            pltpu.VMEM((8, _COLS), jnp.float32),
            pltpu.VMEM((8, _COLS), jnp.float32),
        ],
    )(est2d, gt2d)
    return out[0]


# speedup vs baseline: 1.0986x; 1.0178x over previous
import jax
import jax.numpy as jnp
from jax import lax
from jax.experimental import pallas as pl
from jax.experimental.pallas import tpu as pltpu

VARIANCE_FOCUS = 0.85

_ROWS = 8192
_COLS = 512
_BLK_ROWS = 2048
_GRID = _ROWS // _BLK_ROWS


def _silog_body(est_ref, gt_ref, out_ref, accd_ref, accd2_ref, accn_ref):
    i = pl.program_id(0)

    @pl.when(i == 0)
    def _init():
        accd_ref[...] = jnp.zeros_like(accd_ref)
        accd2_ref[...] = jnp.zeros_like(accd2_ref)
        accn_ref[...] = jnp.zeros_like(accn_ref)

    est = est_ref[...]
    gt = gt_ref[...]
    mask = gt > 1.0
    d = jnp.log(jnp.where(mask, est / gt, 1.0))
    d2 = d * d
    mf = jnp.where(mask, 1.0, 0.0)

    ones = jnp.ones((8, _BLK_ROWS), jnp.float32)
    dims = (((1,), (0,)), ((), ()))
    accd_ref[...] += lax.dot_general(ones, d, dims,
                                     preferred_element_type=jnp.float32)
    accd2_ref[...] += lax.dot_general(ones, d2, dims,
                                      preferred_element_type=jnp.float32)
    accn_ref[...] += lax.dot_general(ones, mf, dims,
                                     preferred_element_type=jnp.float32)

    @pl.when(i == _GRID - 1)
    def _fin():
        sd = jnp.sum(accd_ref[...])
        sd2 = jnp.sum(accd2_ref[...])
        n = jnp.sum(accn_ref[...])
        mean_d = sd / n
        mean_d2 = sd2 / n
        out_ref[0] = jnp.sqrt(mean_d2 - VARIANCE_FOCUS * mean_d * mean_d) * 10.0


def kernel(depth_est, depth_gt):
    est2d = depth_est.reshape(_ROWS, _COLS)
    gt2d = depth_gt.reshape(_ROWS, _COLS)
    out = pl.pallas_call(
        _silog_body,
        grid=(_GRID,),
        in_specs=[
            pl.BlockSpec((_BLK_ROWS, _COLS), lambda i: (i, 0)),
            pl.BlockSpec((_BLK_ROWS, _COLS), lambda i: (i, 0)),
        ],
        out_specs=pl.BlockSpec(memory_space=pltpu.SMEM),
        out_shape=jax.ShapeDtypeStruct((1,), jnp.float32),
        scratch_shapes=[
            pltpu.VMEM((8, _COLS), jnp.float32),
            pltpu.VMEM((8, _COLS), jnp.float32),
            pltpu.VMEM((8, _COLS), jnp.float32),
        ],
    )(est2d, gt2d)
    return out[0]


# speedup vs baseline: 1.1158x; 1.0157x over previous
import jax
import jax.numpy as jnp
from jax import lax
from jax.experimental import pallas as pl
from jax.experimental.pallas import tpu as pltpu

VARIANCE_FOCUS = 0.85

_ROWS = 8192
_COLS = 512
_BLK_ROWS = 2048
_GRID = _ROWS // _BLK_ROWS


def _silog_body(est_ref, gt_ref, out_ref, accd_ref, accd2_ref, accn_ref):
    i = pl.program_id(0)

    @pl.when(i == 0)
    def _init():
        accd_ref[...] = jnp.zeros_like(accd_ref)
        accd2_ref[...] = jnp.zeros_like(accd2_ref)
        accn_ref[0] = 0.0

    est = est_ref[...]
    gt = gt_ref[...]
    mask = gt > 1.0
    d = jnp.log(jnp.where(mask, est / gt, 1.0))
    d2 = d * d

    ones = jnp.ones((8, _BLK_ROWS), jnp.float32)
    dims = (((1,), (0,)), ((), ()))
    accd_ref[...] += lax.dot_general(ones, d, dims,
                                     preferred_element_type=jnp.float32)
    accd2_ref[...] += lax.dot_general(ones, d2, dims,
                                      preferred_element_type=jnp.float32)
    accn_ref[0] += jnp.sum(mask.astype(jnp.float32))

    @pl.when(i == _GRID - 1)
    def _fin():
        sd = jnp.sum(accd_ref[...])
        sd2 = jnp.sum(accd2_ref[...])
        n = 8.0 * accn_ref[0]
        mean_d = sd / n
        mean_d2 = sd2 / n
        out_ref[0] = jnp.sqrt(mean_d2 - VARIANCE_FOCUS * mean_d * mean_d) * 10.0


def kernel(depth_est, depth_gt):
    est2d = depth_est.reshape(_ROWS, _COLS)
    gt2d = depth_gt.reshape(_ROWS, _COLS)
    out = pl.pallas_call(
        _silog_body,
        grid=(_GRID,),
        in_specs=[
            pl.BlockSpec((_BLK_ROWS, _COLS), lambda i: (i, 0)),
            pl.BlockSpec((_BLK_ROWS, _COLS), lambda i: (i, 0)),
        ],
        out_specs=pl.BlockSpec(memory_space=pltpu.SMEM),
        out_shape=jax.ShapeDtypeStruct((1,), jnp.float32),
        scratch_shapes=[
            pltpu.VMEM((8, _COLS), jnp.float32),
            pltpu.VMEM((8, _COLS), jnp.float32),
            pltpu.SMEM((1,), jnp.float32),
        ],
    )(est2d, gt2d)
    return out[0]
